# initial kernel scaffold (unmeasured)
import jax
import jax.numpy as jnp
from jax import lax
from jax.experimental import pallas as pl
from jax.experimental.pallas import tpu as pltpu


def kernel(Q, K, V):
    b, s, h, d = Q.shape
    scale = d ** -0.5

    Qb = Q[0].astype(jnp.bfloat16)
    Kb = K[0].astype(jnp.bfloat16)
    Vb = V[0].astype(jnp.bfloat16)

    def body(q_ref, k_ref, v_ref, o_ref, krem, vrem, send_sems, recv_sems):
        my_x = lax.axis_index("x")
        my_y = lax.axis_index("y")
        my_z = lax.axis_index("z")
        partner = (1 - my_x, my_y, my_z)

        barrier_sem = pltpu.get_barrier_semaphore()
        pl.semaphore_signal(
            barrier_sem, inc=1, device_id=partner,
            device_id_type=pl.DeviceIdType.MESH,
        )
        pl.semaphore_wait(barrier_sem, 1)

        rdma_k = pltpu.make_async_remote_copy(
            src_ref=k_ref, dst_ref=krem,
            send_sem=send_sems.at[0], recv_sem=recv_sems.at[0],
            device_id=partner, device_id_type=pl.DeviceIdType.MESH,
        )
        rdma_v = pltpu.make_async_remote_copy(
            src_ref=v_ref, dst_ref=vrem,
            send_sem=send_sems.at[1], recv_sem=recv_sems.at[1],
            device_id=partner, device_id_type=pl.DeviceIdType.MESH,
        )
        rdma_k.start()
        rdma_v.start()
        rdma_k.wait()
        rdma_v.wait()

        for head in range(h):
            q = q_ref[:, head, :]
            k_full = jnp.concatenate(
                [k_ref[:, head, :], krem[:, head, :]], axis=0)
            v_full = jnp.concatenate(
                [v_ref[:, head, :], vrem[:, head, :]], axis=0)
            s_mat = lax.dot_general(
                q, k_full, (((1,), (1,)), ((), ())),
                preferred_element_type=jnp.float32,
            ) * scale
            m = jnp.max(s_mat, axis=1, keepdims=True)
            p = jnp.exp(s_mat - m)
            l = jnp.sum(p, axis=1, keepdims=True)
            o = lax.dot_general(
                p.astype(jnp.bfloat16), v_full, (((1,), (0,)), ((), ())),
                preferred_element_type=jnp.float32,
            )
            o_ref[:, head, :] = o / l

    out = pl.pallas_call(
        body,
        out_shape=jax.ShapeDtypeStruct((s, h, d), jnp.float32),
        in_specs=[pl.BlockSpec(memory_space=pltpu.VMEM)] * 3,
        out_specs=pl.BlockSpec(memory_space=pltpu.VMEM),
        scratch_shapes=[
            pltpu.VMEM((s, h, d), jnp.bfloat16),
            pltpu.VMEM((s, h, d), jnp.bfloat16),
            pltpu.SemaphoreType.DMA((2,)),
            pltpu.SemaphoreType.DMA((2,)),
        ],
        compiler_params=pltpu.CompilerParams(collective_id=0),
    )(Qb, Kb, Vb)
    return out[None]


# baseline (device time: 215181 ns/iter reference)
import jax
import jax.numpy as jnp
from jax import lax
from jax.experimental import pallas as pl
from jax.experimental.pallas import tpu as pltpu


def kernel(Q, K, V):
    b, s, h, d = Q.shape
    scale = d ** -0.5

    Qb = Q[0].astype(jnp.bfloat16).reshape(s, h * d)
    Kb = K[0].astype(jnp.bfloat16).reshape(s, h * d)
    Vb = V[0].astype(jnp.bfloat16).reshape(s, h * d)

    def comm_body(k_ref, v_ref, krem_ref, vrem_ref, send_sems, recv_sems):
        my_x = lax.axis_index("x")
        my_y = lax.axis_index("y")
        my_z = lax.axis_index("z")
        partner = (1 - my_x, my_y, my_z)

        barrier_sem = pltpu.get_barrier_semaphore()
        pl.semaphore_signal(
            barrier_sem, inc=1, device_id=partner,
            device_id_type=pl.DeviceIdType.MESH,
        )
        pl.semaphore_wait(barrier_sem, 1)

        rdma_k = pltpu.make_async_remote_copy(
            src_ref=k_ref, dst_ref=krem_ref,
            send_sem=send_sems.at[0], recv_sem=recv_sems.at[0],
            device_id=partner, device_id_type=pl.DeviceIdType.MESH,
        )
        rdma_v = pltpu.make_async_remote_copy(
            src_ref=v_ref, dst_ref=vrem_ref,
            send_sem=send_sems.at[1], recv_sem=recv_sems.at[1],
            device_id=partner, device_id_type=pl.DeviceIdType.MESH,
        )
        rdma_k.start()
        rdma_v.start()
        rdma_k.wait()
        rdma_v.wait()

    Krem, Vrem = pl.pallas_call(
        comm_body,
        out_shape=[jax.ShapeDtypeStruct((s, h * d), jnp.bfloat16)] * 2,
        in_specs=[pl.BlockSpec(memory_space=pl.ANY)] * 2,
        out_specs=[pl.BlockSpec(memory_space=pl.ANY)] * 2,
        scratch_shapes=[
            pltpu.SemaphoreType.DMA((2,)),
            pltpu.SemaphoreType.DMA((2,)),
        ],
        compiler_params=pltpu.CompilerParams(collective_id=0),
    )(Kb, Vb)

    def att_body(q_ref, kl_ref, kr_ref, vl_ref, vr_ref, o_ref):
        q = q_ref[:, :]
        k_full = jnp.concatenate([kl_ref[:, :], kr_ref[:, :]], axis=0)
        v_full = jnp.concatenate([vl_ref[:, :], vr_ref[:, :]], axis=0)
        s_mat = lax.dot_general(
            q, k_full, (((1,), (1,)), ((), ())),
            preferred_element_type=jnp.float32,
        ) * scale
        m = jnp.max(s_mat, axis=1, keepdims=True)
        p = jnp.exp(s_mat - m)
        l = jnp.sum(p, axis=1, keepdims=True)
        o = lax.dot_general(
            p.astype(jnp.bfloat16), v_full, (((1,), (0,)), ((), ())),
            preferred_element_type=jnp.float32,
        )
        o_ref[:, :] = o / l

    blk = pl.BlockSpec((s, d), lambda hh: (0, hh))
    out = pl.pallas_call(
        att_body,
        grid=(h,),
        out_shape=jax.ShapeDtypeStruct((s, h * d), jnp.float32),
        in_specs=[blk] * 5,
        out_specs=blk,
    )(Qb, Kb, Krem, Vb, Vrem)
    return out.reshape(1, s, h, d)


# device time: 191967 ns/iter; 1.1209x vs baseline; 1.1209x over previous
import jax
import jax.numpy as jnp
from jax import lax
from jax.experimental import pallas as pl
from jax.experimental.pallas import tpu as pltpu


def kernel(Q, K, V):
    b, s, h, d = Q.shape
    scale = d ** -0.5

    Q2 = Q[0].reshape(s, h * d)
    K2 = K[0].reshape(s, h * d)
    V2 = V[0].reshape(s, h * d)

    def body(q_ref, kl_ref, vl_ref, o_ref,
             kT, vT, krem, vrem, ksend, krecv, vsend, vrecv):
        hh = pl.program_id(0)
        my_x = lax.axis_index("x")
        my_y = lax.axis_index("y")
        my_z = lax.axis_index("z")
        partner = (1 - my_x, my_y, my_z)

        @pl.when(hh == 0)
        def _():
            barrier_sem = pltpu.get_barrier_semaphore()
            pl.semaphore_signal(
                barrier_sem, inc=1, device_id=partner,
                device_id_type=pl.DeviceIdType.MESH,
            )
            pl.semaphore_wait(barrier_sem, 1)

        kT[hh] = kl_ref[:, :].astype(jnp.bfloat16)
        vT[hh] = vl_ref[:, :].astype(jnp.bfloat16)
        rdma_k = pltpu.make_async_remote_copy(
            src_ref=kT.at[hh], dst_ref=krem.at[hh],
            send_sem=ksend.at[hh], recv_sem=krecv.at[hh],
            device_id=partner, device_id_type=pl.DeviceIdType.MESH,
        )
        rdma_v = pltpu.make_async_remote_copy(
            src_ref=vT.at[hh], dst_ref=vrem.at[hh],
            send_sem=vsend.at[hh], recv_sem=vrecv.at[hh],
            device_id=partner, device_id_type=pl.DeviceIdType.MESH,
        )
        rdma_k.start()
        rdma_v.start()

        qb = q_ref[:, :].astype(jnp.bfloat16)
        s1 = lax.dot_general(
            qb, kT[hh], (((1,), (1,)), ((), ())),
            preferred_element_type=jnp.float32,
        ) * scale
        p1 = jnp.exp(s1)
        l1 = jnp.sum(p1, axis=1, keepdims=True)
        u1 = lax.dot_general(
            p1.astype(jnp.bfloat16), vT[hh], (((1,), (0,)), ((), ())),
            preferred_element_type=jnp.float32,
        )

        rdma_k.wait_recv()
        rdma_v.wait_recv()
        s2 = lax.dot_general(
            qb, krem[hh], (((1,), (1,)), ((), ())),
            preferred_element_type=jnp.float32,
        ) * scale
        p2 = jnp.exp(s2)
        l2 = jnp.sum(p2, axis=1, keepdims=True)
        u2 = lax.dot_general(
            p2.astype(jnp.bfloat16), vrem[hh], (((1,), (0,)), ((), ())),
            preferred_element_type=jnp.float32,
        )
        o_ref[:, :] = (u1 + u2) / (l1 + l2)

        rdma_k.wait_send()
        rdma_v.wait_send()

    blk = pl.BlockSpec((s, d), lambda i: (0, i))
    out = pl.pallas_call(
        body,
        grid=(h,),
        out_shape=jax.ShapeDtypeStruct((s, h * d), jnp.float32),
        in_specs=[blk] * 3,
        out_specs=blk,
        scratch_shapes=[
            pltpu.VMEM((h, s, d), jnp.bfloat16),
            pltpu.VMEM((h, s, d), jnp.bfloat16),
            pltpu.VMEM((h, s, d), jnp.bfloat16),
            pltpu.VMEM((h, s, d), jnp.bfloat16),
            pltpu.SemaphoreType.DMA((h,)),
            pltpu.SemaphoreType.DMA((h,)),
            pltpu.SemaphoreType.DMA((h,)),
            pltpu.SemaphoreType.DMA((h,)),
        ],
        compiler_params=pltpu.CompilerParams(
            collective_id=0,
            dimension_semantics=("arbitrary",),
        ),
    )(Q2, K2, V2)
    return out.reshape(1, s, h, d)


# device time: 123222 ns/iter; 1.7463x vs baseline; 1.5579x over previous
import jax
import jax.numpy as jnp
from jax import lax
from jax.experimental import pallas as pl
from jax.experimental.pallas import tpu as pltpu

LAG = 2
RING = 4


def kernel(Q, K, V):
    b, s, h, d = Q.shape
    scale = d ** -0.5

    Q2 = Q[0].reshape(s, h * d)
    K2 = K[0].reshape(s, h * d)
    V2 = V[0].reshape(s, h * d)

    def body(q_ref, kl_ref, vl_ref, o_ref,
             qT, kT, vT, krem, vrem, uring, lring,
             ksend, krecv, vsend, vrecv):
        i = pl.program_id(0)
        my_x = lax.axis_index("x")
        my_y = lax.axis_index("y")
        my_z = lax.axis_index("z")
        partner = (1 - my_x, my_y, my_z)

        @pl.when(i == 0)
        def _():
            barrier_sem = pltpu.get_barrier_semaphore()
            pl.semaphore_signal(
                barrier_sem, inc=1, device_id=partner,
                device_id_type=pl.DeviceIdType.MESH,
            )
            pl.semaphore_wait(barrier_sem, 1)

        @pl.when(i < h)
        def _():
            qT[i] = q_ref[:, :].astype(jnp.bfloat16)
            kT[i] = kl_ref[:, :].astype(jnp.bfloat16)
            vT[i] = vl_ref[:, :].astype(jnp.bfloat16)
            rdma_k = pltpu.make_async_remote_copy(
                src_ref=kT.at[i], dst_ref=krem.at[i],
                send_sem=ksend.at[i], recv_sem=krecv.at[i],
                device_id=partner, device_id_type=pl.DeviceIdType.MESH,
            )
            rdma_v = pltpu.make_async_remote_copy(
                src_ref=vT.at[i], dst_ref=vrem.at[i],
                send_sem=vsend.at[i], recv_sem=vrecv.at[i],
                device_id=partner, device_id_type=pl.DeviceIdType.MESH,
            )
            rdma_k.start()
            rdma_v.start()

            r = lax.rem(i, RING)
            s1 = lax.dot_general(
                qT[i], kT[i], (((1,), (1,)), ((), ())),
                preferred_element_type=jnp.float32,
            ) * scale
            p1 = jnp.exp(s1)
            l1 = jnp.sum(p1, axis=1, keepdims=True)
            uring[r] = lax.dot_general(
                p1.astype(jnp.bfloat16), vT[i], (((1,), (0,)), ((), ())),
                preferred_element_type=jnp.float32,
            )
            lring[r] = jnp.broadcast_to(l1, (s, 128))

        @pl.when(i >= LAG)
        def _():
            m = i - LAG
            rm = lax.rem(m, RING)
            rdma_k = pltpu.make_async_remote_copy(
                src_ref=kT.at[m], dst_ref=krem.at[m],
                send_sem=ksend.at[m], recv_sem=krecv.at[m],
                device_id=partner, device_id_type=pl.DeviceIdType.MESH,
            )
            rdma_v = pltpu.make_async_remote_copy(
                src_ref=vT.at[m], dst_ref=vrem.at[m],
                send_sem=vsend.at[m], recv_sem=vrecv.at[m],
                device_id=partner, device_id_type=pl.DeviceIdType.MESH,
            )
            rdma_k.wait_recv()
            rdma_v.wait_recv()
            s2 = lax.dot_general(
                qT[m], krem[m], (((1,), (1,)), ((), ())),
                preferred_element_type=jnp.float32,
            ) * scale
            p2 = jnp.exp(s2)
            l2 = jnp.sum(p2, axis=1, keepdims=True)
            u2 = lax.dot_general(
                p2.astype(jnp.bfloat16), vrem[m], (((1,), (0,)), ((), ())),
                preferred_element_type=jnp.float32,
            )
            o_ref[:, :] = (uring[rm] + u2) / (lring[rm][:, 0:1] + l2)
            rdma_k.wait_send()
            rdma_v.wait_send()

    in_blk = pl.BlockSpec((s, d), lambda i: (0, jnp.minimum(i, h - 1)))
    out_blk = pl.BlockSpec((s, d), lambda i: (0, jnp.maximum(i - LAG, 0)))
    out = pl.pallas_call(
        body,
        grid=(h + LAG,),
        out_shape=jax.ShapeDtypeStruct((s, h * d), jnp.float32),
        in_specs=[in_blk] * 3,
        out_specs=out_blk,
        scratch_shapes=[
            pltpu.VMEM((h, s, d), jnp.bfloat16),
            pltpu.VMEM((h, s, d), jnp.bfloat16),
            pltpu.VMEM((h, s, d), jnp.bfloat16),
            pltpu.VMEM((h, s, d), jnp.bfloat16),
            pltpu.VMEM((h, s, d), jnp.bfloat16),
            pltpu.VMEM((RING, s, d), jnp.float32),
            pltpu.VMEM((RING, s, 128), jnp.float32),
            pltpu.SemaphoreType.DMA((h,)),
            pltpu.SemaphoreType.DMA((h,)),
            pltpu.SemaphoreType.DMA((h,)),
            pltpu.SemaphoreType.DMA((h,)),
        ],
        compiler_params=pltpu.CompilerParams(
            collective_id=0,
            dimension_semantics=("arbitrary",),
        ),
    )(Q2, K2, V2)
    return out.reshape(1, s, h, d)


# device time: 104806 ns/iter; 2.0531x vs baseline; 1.1757x over previous
import jax
import jax.numpy as jnp
from jax import lax
from jax.experimental import pallas as pl
from jax.experimental.pallas import tpu as pltpu

LAG = 2
RING = 4


def kernel(Q, K, V):
    b, s, h, d = Q.shape
    scale = d ** -0.5

    def body(q_hbm, k_hbm, v_hbm, o_hbm,
             qs, ks, vs, ostage, qT, kT, vT, krem, vrem, uring, lring,
             in_sems, o_sems, ksend, krecv, vsend, vrecv):
        i = pl.program_id(0)
        my_x = lax.axis_index("x")
        my_y = lax.axis_index("y")
        my_z = lax.axis_index("z")
        partner = (1 - my_x, my_y, my_z)

        def in_dma(j, slot):
            return [
                pltpu.make_async_copy(
                    ref.at[0, :, j, :], stage.at[slot], in_sems.at[slot, t])
                for t, (ref, stage) in enumerate(
                    [(q_hbm, qs), (k_hbm, ks), (v_hbm, vs)])
            ]

        @pl.when(i == 0)
        def _():
            barrier_sem = pltpu.get_barrier_semaphore()
            pl.semaphore_signal(
                barrier_sem, inc=1, device_id=partner,
                device_id_type=pl.DeviceIdType.MESH,
            )
            pl.semaphore_wait(barrier_sem, 1)
            for c in in_dma(0, 0):
                c.start()

        @pl.when(i < h)
        def _():
            slot = lax.rem(i, 2)
            for c in in_dma(i, slot):
                c.wait()

            @pl.when(i + 1 < h)
            def _():
                for c in in_dma(i + 1, lax.rem(i + 1, 2)):
                    c.start()

            qT[i] = qs[slot].astype(jnp.bfloat16)
            kT[i] = ks[slot].astype(jnp.bfloat16)
            vT[i] = vs[slot].astype(jnp.bfloat16)
            rdma_k = pltpu.make_async_remote_copy(
                src_ref=kT.at[i], dst_ref=krem.at[i],
                send_sem=ksend.at[i], recv_sem=krecv.at[i],
                device_id=partner, device_id_type=pl.DeviceIdType.MESH,
            )
            rdma_v = pltpu.make_async_remote_copy(
                src_ref=vT.at[i], dst_ref=vrem.at[i],
                send_sem=vsend.at[i], recv_sem=vrecv.at[i],
                device_id=partner, device_id_type=pl.DeviceIdType.MESH,
            )
            rdma_k.start()
            rdma_v.start()

            r = lax.rem(i, RING)
            s1 = lax.dot_general(
                qT[i], kT[i], (((1,), (1,)), ((), ())),
                preferred_element_type=jnp.float32,
            ) * scale
            p1 = jnp.exp(s1)
            l1 = jnp.sum(p1, axis=1, keepdims=True)
            uring[r] = lax.dot_general(
                p1.astype(jnp.bfloat16), vT[i], (((1,), (0,)), ((), ())),
                preferred_element_type=jnp.float32,
            )
            lring[r] = jnp.broadcast_to(l1, (s, 128))

        @pl.when(i >= LAG)
        def _():
            m = i - LAG
            sm = lax.rem(m, 2)
            rm = lax.rem(m, RING)
            rdma_k = pltpu.make_async_remote_copy(
                src_ref=kT.at[m], dst_ref=krem.at[m],
                send_sem=ksend.at[m], recv_sem=krecv.at[m],
                device_id=partner, device_id_type=pl.DeviceIdType.MESH,
            )
            rdma_v = pltpu.make_async_remote_copy(
                src_ref=vT.at[m], dst_ref=vrem.at[m],
                send_sem=vsend.at[m], recv_sem=vrecv.at[m],
                device_id=partner, device_id_type=pl.DeviceIdType.MESH,
            )
            rdma_k.wait_recv()
            rdma_v.wait_recv()
            s2 = lax.dot_general(
                qT[m], krem[m], (((1,), (1,)), ((), ())),
                preferred_element_type=jnp.float32,
            ) * scale
            p2 = jnp.exp(s2)
            l2 = jnp.sum(p2, axis=1, keepdims=True)
            u2 = lax.dot_general(
                p2.astype(jnp.bfloat16), vrem[m], (((1,), (0,)), ((), ())),
                preferred_element_type=jnp.float32,
            )

            def o_dma(j, slot):
                return pltpu.make_async_copy(
                    ostage.at[slot], o_hbm.at[0, :, j, :], o_sems.at[slot])

            @pl.when(m >= 2)
            def _():
                o_dma(m - 2, sm).wait()

            ostage[sm] = (uring[rm] + u2) / (lring[rm][:, 0:1] + l2)
            o_dma(m, sm).start()

            rdma_k.wait_send()
            rdma_v.wait_send()

            @pl.when(m == h - 1)
            def _():
                o_dma(m - 1, lax.rem(m - 1, 2)).wait()
                o_dma(m, sm).wait()

    out = pl.pallas_call(
        body,
        grid=(h + LAG,),
        out_shape=jax.ShapeDtypeStruct((b, s, h, d), jnp.float32),
        in_specs=[pl.BlockSpec(memory_space=pl.ANY)] * 3,
        out_specs=pl.BlockSpec(memory_space=pl.ANY),
        scratch_shapes=[
            pltpu.VMEM((2, s, d), jnp.float32),
            pltpu.VMEM((2, s, d), jnp.float32),
            pltpu.VMEM((2, s, d), jnp.float32),
            pltpu.VMEM((2, s, d), jnp.float32),
            pltpu.VMEM((h, s, d), jnp.bfloat16),
            pltpu.VMEM((h, s, d), jnp.bfloat16),
            pltpu.VMEM((h, s, d), jnp.bfloat16),
            pltpu.VMEM((h, s, d), jnp.bfloat16),
            pltpu.VMEM((h, s, d), jnp.bfloat16),
            pltpu.VMEM((RING, s, d), jnp.float32),
            pltpu.VMEM((RING, s, 128), jnp.float32),
            pltpu.SemaphoreType.DMA((2, 3)),
            pltpu.SemaphoreType.DMA((2,)),
            pltpu.SemaphoreType.DMA((h,)),
            pltpu.SemaphoreType.DMA((h,)),
            pltpu.SemaphoreType.DMA((h,)),
            pltpu.SemaphoreType.DMA((h,)),
        ],
        compiler_params=pltpu.CompilerParams(
            collective_id=0,
            dimension_semantics=("arbitrary",),
        ),
    )(Q, K, V)
    return out
